# Initial kernel scaffold; baseline (speedup 1.0000x reference)
import jax
import jax.numpy as jnp
from jax.experimental import pallas as pl


def kernel(pred_logits, pred_boxes, target_sizes):
    raise NotImplementedError("write your pallas kernel here")


# speedup vs baseline: 13.4470x; 13.4470x over previous
import jax
import jax.numpy as jnp
from jax.experimental import pallas as pl

_NSEL = 300


def _rowmax_body(prob_ref, out_ref):
    out_ref[0, 0, :] = jnp.max(prob_ref[0], axis=-1)


def _row_max(probs):
    B, N, C = probs.shape
    out = pl.pallas_call(
        _rowmax_body,
        grid=(B,),
        in_specs=[pl.BlockSpec((1, N, C), lambda b: (b, 0, 0))],
        out_specs=pl.BlockSpec((1, 1, N), lambda b: (b, 0, 0)),
        out_shape=jax.ShapeDtypeStruct((B, 1, N), probs.dtype),
    )(probs)
    return out.reshape(B, N)


def kernel(pred_logits, pred_boxes, target_sizes):
    B, N, C = pred_logits.shape
    probs = jax.nn.sigmoid(pred_logits)
    row_max = _row_max(probs)

    _, rows = jax.lax.top_k(row_max, _NSEL)
    rows = jnp.sort(rows, axis=-1)
    cand = jnp.take_along_axis(probs, rows[:, :, None], axis=1)
    scores, pos = jax.lax.top_k(cand.reshape(B, _NSEL * C), _NSEL)
    labels = pos % C
    slot = pos // C
    box_rows = jnp.take_along_axis(rows, slot, axis=1)

    b4 = jnp.take_along_axis(pred_boxes, box_rows[:, :, None], axis=1)
    cx, cy, w, h = b4[..., 0], b4[..., 1], b4[..., 2], b4[..., 3]
    boxes = jnp.stack([cx - 0.5 * w, cy - 0.5 * h, cx + 0.5 * w, cy + 0.5 * h], axis=-1)
    img_h = target_sizes[:, 0].astype(jnp.float32)
    img_w = target_sizes[:, 1].astype(jnp.float32)
    scale = jnp.stack([img_w, img_h, img_w, img_h], axis=1)
    boxes = boxes * scale[:, None, :]
    return boxes, scores, labels


# speedup vs baseline: 17.5383x; 1.3043x over previous
import jax
import jax.numpy as jnp
from jax.experimental import pallas as pl

_NSEL = 300


def _rowmax_body(logit_ref, out_ref):
    out_ref[0, 0, :] = jnp.max(jax.nn.sigmoid(logit_ref[0]), axis=-1)


def _row_max(probs):
    B, N, C = probs.shape
    out = pl.pallas_call(
        _rowmax_body,
        grid=(B,),
        in_specs=[pl.BlockSpec((1, N, C), lambda b: (b, 0, 0))],
        out_specs=pl.BlockSpec((1, 1, N), lambda b: (b, 0, 0)),
        out_shape=jax.ShapeDtypeStruct((B, 1, N), probs.dtype),
    )(probs)
    return out.reshape(B, N)


def kernel(pred_logits, pred_boxes, target_sizes):
    B, N, C = pred_logits.shape
    row_max = _row_max(pred_logits)

    _, rows = jax.lax.top_k(row_max, _NSEL)
    rows = jnp.sort(rows, axis=-1)
    cand = jax.nn.sigmoid(
        jnp.take_along_axis(pred_logits, rows[:, :, None], axis=1))
    scores, pos = jax.lax.top_k(cand.reshape(B, _NSEL * C), _NSEL)
    labels = pos % C
    slot = pos // C
    box_rows = jnp.take_along_axis(rows, slot, axis=1)

    b4 = jnp.take_along_axis(pred_boxes, box_rows[:, :, None], axis=1)
    cx, cy, w, h = b4[..., 0], b4[..., 1], b4[..., 2], b4[..., 3]
    boxes = jnp.stack([cx - 0.5 * w, cy - 0.5 * h, cx + 0.5 * w, cy + 0.5 * h], axis=-1)
    img_h = target_sizes[:, 0].astype(jnp.float32)
    img_w = target_sizes[:, 1].astype(jnp.float32)
    scale = jnp.stack([img_w, img_h, img_w, img_h], axis=1)
    boxes = boxes * scale[:, None, :]
    return boxes, scores, labels
